# baseline (speedup 1.0000x reference)
import functools

import jax
import jax.numpy as jnp
from jax import lax
from jax.experimental import pallas as pl
from jax.experimental.pallas import tpu as pltpu
from jax.experimental.pallas import tpu_sc as plsc

_N_NODES = 10000
_N_EDGES = 320000
_D = 128
_EA_DIM = 16

_NC = 2
_NS = 16
_NW = _NC * _NS
_K = 5
_EPK = _N_EDGES // _K
_EDGES_PER_W = _EPK // _NW
_C = 200
_CHUNKS_PER_W = _EDGES_PER_W // _C
_EB = 2000


def _tc_tables(x, w1ab):
    def body(x_ref, w_ref, out_ref):
        out_ref[...] = jnp.dot(x_ref[...], w_ref[0],
                               preferred_element_type=jnp.float32)

    return pl.pallas_call(
        body,
        grid=(2,),
        in_specs=[
            pl.BlockSpec((_N_NODES, _D), lambda t: (0, 0)),
            pl.BlockSpec((1, _D, _D), lambda t: (t, 0, 0)),
        ],
        out_specs=pl.BlockSpec((_N_NODES, _D), lambda t: (t, 0)),
        out_shape=jax.ShapeDtypeStruct((2 * _N_NODES, _D), jnp.float32),
    )(x, w1ab)


def _alloc_dest():
    def body(dest_ref):
        del dest_ref

    return pl.pallas_call(
        body,
        out_specs=pl.BlockSpec(memory_space=pl.ANY),
        out_shape=jax.ShapeDtypeStruct((_N_EDGES, _D), jnp.float32),
    )()


def _sc_gather_sum(table, src, tgt_off, slice_k):
    mesh = plsc.VectorSubcoreMesh(core_axis_name="c", subcore_axis_name="s",
                                  num_cores=_NC, num_subcores=_NS)
    n_chunks = _CHUNKS_PER_W

    @functools.partial(
        pl.kernel,
        out_type=jax.ShapeDtypeStruct((_EPK, _D), jnp.float32),
        mesh=mesh,
        scratch_types=[
            pltpu.VMEM((_C,), jnp.int32),
            pltpu.VMEM((_C,), jnp.int32),
            pltpu.VMEM((_C,), jnp.int32),
            pltpu.VMEM((_C,), jnp.int32),
            pltpu.VMEM((_C, _D), jnp.float32),
            pltpu.VMEM((_C, _D), jnp.float32),
            pltpu.VMEM((_C, _D), jnp.float32),
            pltpu.VMEM((_C, _D), jnp.float32),
            pltpu.SemaphoreType.DMA,
            pltpu.SemaphoreType.DMA,
            pltpu.SemaphoreType.DMA,
            pltpu.SemaphoreType.DMA,
            pltpu.SemaphoreType.DMA,
            pltpu.SemaphoreType.DMA,
        ],
    )
    def k(table_hbm, src_hbm, tgt_hbm, out_hbm,
          ia0, ib0, ia1, ib1, a0, b0, a1, b1, g0, g1, s0, s1, w0, w1):
        wid = lax.axis_index("s") * _NC + lax.axis_index("c")
        in_base = slice_k * _EPK + wid * _EDGES_PER_W
        out_base = wid * _EDGES_PER_W
        IA = (ia0, ia1)
        IB = (ib0, ib1)
        A = (a0, a1)
        B = (b0, b1)
        G = (g0, g1)
        S = (s0, s1)
        W = (w0, w1)

        def fire_idx(j, s):
            off = pl.multiple_of(in_base + j * _C, 8)
            pltpu.async_copy(src_hbm.at[pl.ds(off, _C)], IA[s], S[s])
            pltpu.async_copy(tgt_hbm.at[pl.ds(off, _C)], IB[s], S[s])

        def wait_idx(s):
            pltpu.make_async_copy(src_hbm.at[pl.ds(0, _C)], IA[s], S[s]).wait()
            pltpu.make_async_copy(tgt_hbm.at[pl.ds(0, _C)], IB[s], S[s]).wait()

        def fire_gather(s):
            pltpu.async_copy(table_hbm.at[IA[s]], A[s], G[s])
            pltpu.async_copy(table_hbm.at[IB[s]], B[s], G[s])

        def wait_gather(s):
            pltpu.make_async_copy(table_hbm.at[IA[s]], A[s], G[s]).wait()
            pltpu.make_async_copy(table_hbm.at[IB[s]], B[s], G[s]).wait()

        def fire_write(j, s):
            off = pl.multiple_of(out_base + j * _C, 8)
            pltpu.async_copy(A[s], out_hbm.at[pl.ds(off, _C)], W[s])

        def wait_write(s):
            pltpu.make_async_copy(A[s], out_hbm.at[pl.ds(0, _C)], W[s]).wait()

        def add_rows(s):
            acc = A[s]
            rb = B[s]

            @plsc.parallel_loop(0, _C, unroll=4)
            def _(r):
                for q in range(_D // 16):
                    sl = pl.ds(q * 16, 16)
                    acc[r, sl] = acc[r, sl] + rb[r, sl]

        def phase(j, s):
            wait_gather(s)

            @pl.when(j <= n_chunks - 2)
            def _():
                wait_idx(s ^ 1)

                @pl.when(j >= 1)
                def _():
                    wait_write(s ^ 1)

                fire_gather(s ^ 1)

                @pl.when(j <= n_chunks - 3)
                def _():
                    fire_idx(j + 2, s)

            add_rows(s)
            fire_write(j, s)

        fire_idx(0, 0)
        wait_idx(0)
        fire_gather(0)
        fire_idx(1, 1)

        def pair(i, carry):
            phase(2 * i, 0)
            phase(2 * i + 1, 1)
            return carry

        lax.fori_loop(0, n_chunks // 2, pair, 0)
        if n_chunks % 2:
            phase(jnp.int32(n_chunks - 1), 0)
        wait_write(0)
        wait_write(1)

    return k(table, src, tgt_off)


def _mlp_body(dst_ref, g_ref, ea_ref, w1c_ref, b1_ref, w2_ref, b2_ref,
              out_ref):
    del dst_ref
    pre = jnp.dot(ea_ref[...], w1c_ref[...],
                  preferred_element_type=jnp.float32)
    h = jnp.maximum(pre + g_ref[...] + b1_ref[...], 0.0)
    out_ref[...] = jnp.dot(h, w2_ref[...],
                           preferred_element_type=jnp.float32) + b2_ref[...]


def _tc_mlp_slice(dest, gsum, ea_slice, w1c, b1, w2, b2, slice_k):
    blk0 = slice_k * (_EPK // _EB)
    in_specs = [
        pl.BlockSpec((_EB, _D), lambda i: (i, 0)),
        pl.BlockSpec((_EB, _EA_DIM), lambda i: (i, 0)),
        pl.BlockSpec((_EA_DIM, _D), lambda i: (0, 0)),
        pl.BlockSpec((1, _D), lambda i: (0, 0)),
        pl.BlockSpec((_D, _D), lambda i: (0, 0)),
        pl.BlockSpec((1, _D), lambda i: (0, 0)),
    ]
    out_spec = pl.BlockSpec((_EB, _D), lambda i: (blk0 + i, 0))
    out_shape = jax.ShapeDtypeStruct((_N_EDGES, _D), jnp.float32)
    return pl.pallas_call(
        _mlp_body,
        grid=(_EPK // _EB,),
        in_specs=[pl.BlockSpec(memory_space=pl.ANY)] + in_specs,
        out_specs=out_spec,
        out_shape=out_shape,
        input_output_aliases={0: 0},
    )(dest, gsum, ea_slice, w1c, b1, w2, b2)


def kernel(x, edge_index, edge_attr, W1, b1, W2, b2):
    src = edge_index[0].astype(jnp.int32)
    tgt_off = edge_index[1].astype(jnp.int32) + _N_NODES
    w1ab = W1[: 2 * _D].reshape(2, _D, _D)
    w1c = W1[2 * _D:]
    b1r = b1.reshape(1, _D)
    b2r = b2.reshape(1, _D)
    table = _tc_tables(x, w1ab)
    dest = _alloc_dest()
    gsums = [_sc_gather_sum(table, src, tgt_off, k) for k in range(_K)]
    for k in range(_K):
        ea_k = jax.lax.slice_in_dim(edge_attr, k * _EPK, (k + 1) * _EPK)
        dest = _tc_mlp_slice(dest, gsums[k], ea_k, w1c, b1r, W2, b2r, k)
    return dest


# speedup vs baseline: 1.0437x; 1.0437x over previous
import functools

import jax
import jax.numpy as jnp
from jax import lax
from jax.experimental import pallas as pl
from jax.experimental.pallas import tpu as pltpu
from jax.experimental.pallas import tpu_sc as plsc

_N_NODES = 10000
_N_EDGES = 320000
_D = 128
_EA_DIM = 16

_NC = 2
_NS = 16
_NW = _NC * _NS
_K = 5
_EPK = _N_EDGES // _K
_EDGES_PER_W = _EPK // _NW
_C = 200
_CHUNKS_PER_W = _EDGES_PER_W // _C
_EB = 2000


def _tc_tables(x, w1ab):
    def body(x_ref, w_ref, out_ref):
        out_ref[...] = jnp.dot(x_ref[...], w_ref[0],
                               preferred_element_type=jnp.float32)

    return pl.pallas_call(
        body,
        grid=(2,),
        in_specs=[
            pl.BlockSpec((_N_NODES, _D), lambda t: (0, 0)),
            pl.BlockSpec((1, _D, _D), lambda t: (t, 0, 0)),
        ],
        out_specs=pl.BlockSpec((_N_NODES, _D), lambda t: (t, 0)),
        out_shape=jax.ShapeDtypeStruct((2 * _N_NODES, _D), jnp.float32),
    )(x, w1ab)


def _alloc_dest():
    def body(dest_ref):
        del dest_ref

    return pl.pallas_call(
        body,
        out_specs=pl.BlockSpec(memory_space=pl.ANY),
        out_shape=jax.ShapeDtypeStruct((_N_EDGES, _D), jnp.float32),
    )()


def _sc_gather_sum(table, src, tgt_off, slice_k):
    mesh = plsc.VectorSubcoreMesh(core_axis_name="c", subcore_axis_name="s",
                                  num_cores=_NC, num_subcores=_NS)
    n_chunks = _CHUNKS_PER_W

    @functools.partial(
        pl.kernel,
        out_type=jax.ShapeDtypeStruct((_EPK, _D), jnp.float32),
        mesh=mesh,
        scratch_types=[
            pltpu.VMEM((_C,), jnp.int32),
            pltpu.VMEM((_C,), jnp.int32),
            pltpu.VMEM((_C,), jnp.int32),
            pltpu.VMEM((_C,), jnp.int32),
            pltpu.VMEM((_C, _D), jnp.float32),
            pltpu.VMEM((_C, _D), jnp.float32),
            pltpu.VMEM((_C, _D), jnp.float32),
            pltpu.VMEM((_C, _D), jnp.float32),
            pltpu.SemaphoreType.DMA,
            pltpu.SemaphoreType.DMA,
            pltpu.SemaphoreType.DMA,
            pltpu.SemaphoreType.DMA,
            pltpu.SemaphoreType.DMA,
            pltpu.SemaphoreType.DMA,
        ],
    )
    def k(table_hbm, src_hbm, tgt_hbm, out_hbm,
          ia0, ib0, ia1, ib1, a0, b0, a1, b1, g0, g1, s0, s1, w0, w1):
        wid = lax.axis_index("s") * _NC + lax.axis_index("c")
        in_base = slice_k * _EPK + wid * _EDGES_PER_W
        out_base = wid * _EDGES_PER_W
        IA = (ia0, ia1)
        IB = (ib0, ib1)
        A = (a0, a1)
        B = (b0, b1)
        G = (g0, g1)
        S = (s0, s1)
        W = (w0, w1)

        def fire_idx(j, s):
            off = pl.multiple_of(in_base + j * _C, 8)
            pltpu.async_copy(src_hbm.at[pl.ds(off, _C)], IA[s], S[s])
            pltpu.async_copy(tgt_hbm.at[pl.ds(off, _C)], IB[s], S[s])

        def wait_idx(s):
            pltpu.make_async_copy(src_hbm.at[pl.ds(0, _C)], IA[s], S[s]).wait()
            pltpu.make_async_copy(tgt_hbm.at[pl.ds(0, _C)], IB[s], S[s]).wait()

        def fire_gather(s):
            pltpu.async_copy(table_hbm.at[IA[s]], A[s], G[s])
            pltpu.async_copy(table_hbm.at[IB[s]], B[s], G[s])

        def wait_gather(s):
            pltpu.make_async_copy(table_hbm.at[IA[s]], A[s], G[s]).wait()
            pltpu.make_async_copy(table_hbm.at[IB[s]], B[s], G[s]).wait()

        def fire_write(j, s):
            off = pl.multiple_of(out_base + j * _C, 8)
            pltpu.async_copy(A[s], out_hbm.at[pl.ds(off, _C)], W[s])

        def wait_write(s):
            pltpu.make_async_copy(A[s], out_hbm.at[pl.ds(0, _C)], W[s]).wait()

        def add_rows(s):
            acc = A[s]
            rb = B[s]

            @plsc.parallel_loop(0, _C, unroll=4)
            def _(r):
                for q in range(_D // 16):
                    sl = pl.ds(q * 16, 16)
                    acc[r, sl] = acc[r, sl] + rb[r, sl]

        def phase(j, s):
            wait_gather(s)

            @pl.when(j <= n_chunks - 2)
            def _():
                wait_idx(s ^ 1)

                @pl.when(j >= 1)
                def _():
                    wait_write(s ^ 1)

                fire_gather(s ^ 1)

                @pl.when(j <= n_chunks - 3)
                def _():
                    fire_idx(j + 2, s)

            add_rows(s)
            fire_write(j, s)

        fire_idx(0, 0)
        wait_idx(0)
        fire_gather(0)
        fire_idx(1, 1)

        def pair(i, carry):
            phase(2 * i, 0)
            phase(2 * i + 1, 1)
            return carry

        lax.fori_loop(0, n_chunks // 2, pair, 0)
        if n_chunks % 2:
            phase(jnp.int32(n_chunks - 1), 0)
        wait_write(0)
        wait_write(1)

    return k(table, src, tgt_off)


def _mlp_body(dst_ref, g_ref, ea_ref, w1c_ref, b1_ref, w2_ref, b2_ref,
              out_ref):
    del dst_ref
    pre = jnp.dot(ea_ref[...], w1c_ref[...],
                  preferred_element_type=jnp.float32)
    h = jnp.maximum(pre + g_ref[...] + b1_ref[...], 0.0)
    out_ref[...] = jnp.dot(h, w2_ref[...],
                           preferred_element_type=jnp.float32) + b2_ref[...]


def _tc_mlp_slice(dest, gsum, edge_attr, w1c, b1, w2, b2, slice_k):
    blk0 = slice_k * (_EPK // _EB)
    in_specs = [
        pl.BlockSpec((_EB, _D), lambda i: (i, 0)),
        pl.BlockSpec((_EB, _EA_DIM), lambda i: (blk0 + i, 0)),
        pl.BlockSpec((_EA_DIM, _D), lambda i: (0, 0)),
        pl.BlockSpec((1, _D), lambda i: (0, 0)),
        pl.BlockSpec((_D, _D), lambda i: (0, 0)),
        pl.BlockSpec((1, _D), lambda i: (0, 0)),
    ]
    out_spec = pl.BlockSpec((_EB, _D), lambda i: (blk0 + i, 0))
    out_shape = jax.ShapeDtypeStruct((_N_EDGES, _D), jnp.float32)
    return pl.pallas_call(
        _mlp_body,
        grid=(_EPK // _EB,),
        in_specs=[pl.BlockSpec(memory_space=pl.ANY)] + in_specs,
        out_specs=out_spec,
        out_shape=out_shape,
        input_output_aliases={0: 0},
    )(dest, gsum, edge_attr, w1c, b1, w2, b2)


def kernel(x, edge_index, edge_attr, W1, b1, W2, b2):
    src = edge_index[0].astype(jnp.int32)
    tgt_off = edge_index[1].astype(jnp.int32) + _N_NODES
    w1ab = W1[: 2 * _D].reshape(2, _D, _D)
    w1c = W1[2 * _D:]
    b1r = b1.reshape(1, _D)
    b2r = b2.reshape(1, _D)
    table = _tc_tables(x, w1ab)
    dest = _alloc_dest()
    gsums = [_sc_gather_sum(table, src, tgt_off, k) for k in range(_K)]
    for k in range(_K):
        dest = _tc_mlp_slice(dest, gsums[k], edge_attr, w1c, b1r, W2, b2r, k)
    return dest


# speedup vs baseline: 1.1162x; 1.0695x over previous
import functools

import jax
import jax.numpy as jnp
from jax import lax
from jax.experimental import pallas as pl
from jax.experimental.pallas import tpu as pltpu
from jax.experimental.pallas import tpu_sc as plsc

_N_NODES = 10000
_N_EDGES = 320000
_D = 128
_EA_DIM = 16

_NC = 2
_NS = 16
_NW = _NC * _NS
_K = 5
_EPK = _N_EDGES // _K
_EDGES_PER_W = _EPK // _NW
_C = 200
_CHUNKS_PER_W = _EDGES_PER_W // _C
_EB = 4000


def _tc_tables(x, w1ab):
    def body(x_ref, w_ref, out_ref):
        out_ref[...] = jnp.dot(x_ref[...], w_ref[0],
                               preferred_element_type=jnp.float32)

    return pl.pallas_call(
        body,
        grid=(2,),
        in_specs=[
            pl.BlockSpec((_N_NODES, _D), lambda t: (0, 0)),
            pl.BlockSpec((1, _D, _D), lambda t: (t, 0, 0)),
        ],
        out_specs=pl.BlockSpec((_N_NODES, _D), lambda t: (t, 0)),
        out_shape=jax.ShapeDtypeStruct((2 * _N_NODES, _D), jnp.float32),
    )(x, w1ab)


def _alloc_dest():
    def body(dest_ref):
        del dest_ref

    return pl.pallas_call(
        body,
        out_specs=pl.BlockSpec(memory_space=pl.ANY),
        out_shape=jax.ShapeDtypeStruct((_N_EDGES, _D), jnp.float32),
    )()


def _sc_gather_sum(table, src, tgt_off, slice_k):
    mesh = plsc.VectorSubcoreMesh(core_axis_name="c", subcore_axis_name="s",
                                  num_cores=_NC, num_subcores=_NS)
    n_chunks = _CHUNKS_PER_W

    @functools.partial(
        pl.kernel,
        out_type=jax.ShapeDtypeStruct((_EPK, _D), jnp.float32),
        mesh=mesh,
        scratch_types=[
            pltpu.VMEM((_C,), jnp.int32),
            pltpu.VMEM((_C,), jnp.int32),
            pltpu.VMEM((_C,), jnp.int32),
            pltpu.VMEM((_C,), jnp.int32),
            pltpu.VMEM((_C, _D), jnp.float32),
            pltpu.VMEM((_C, _D), jnp.float32),
            pltpu.VMEM((_C, _D), jnp.float32),
            pltpu.VMEM((_C, _D), jnp.float32),
            pltpu.SemaphoreType.DMA,
            pltpu.SemaphoreType.DMA,
            pltpu.SemaphoreType.DMA,
            pltpu.SemaphoreType.DMA,
            pltpu.SemaphoreType.DMA,
            pltpu.SemaphoreType.DMA,
        ],
    )
    def k(table_hbm, src_hbm, tgt_hbm, out_hbm,
          ia0, ib0, ia1, ib1, a0, b0, a1, b1, g0, g1, s0, s1, w0, w1):
        wid = lax.axis_index("s") * _NC + lax.axis_index("c")
        in_base = slice_k * _EPK + wid * _EDGES_PER_W
        out_base = wid * _EDGES_PER_W
        IA = (ia0, ia1)
        IB = (ib0, ib1)
        A = (a0, a1)
        B = (b0, b1)
        G = (g0, g1)
        S = (s0, s1)
        W = (w0, w1)

        def fire_idx(j, s):
            off = pl.multiple_of(in_base + j * _C, 8)
            pltpu.async_copy(src_hbm.at[pl.ds(off, _C)], IA[s], S[s])
            pltpu.async_copy(tgt_hbm.at[pl.ds(off, _C)], IB[s], S[s])

        def wait_idx(s):
            pltpu.make_async_copy(src_hbm.at[pl.ds(0, _C)], IA[s], S[s]).wait()
            pltpu.make_async_copy(tgt_hbm.at[pl.ds(0, _C)], IB[s], S[s]).wait()

        def fire_gather(s):
            pltpu.async_copy(table_hbm.at[IA[s]], A[s], G[s])
            pltpu.async_copy(table_hbm.at[IB[s]], B[s], G[s])

        def wait_gather(s):
            pltpu.make_async_copy(table_hbm.at[IA[s]], A[s], G[s]).wait()
            pltpu.make_async_copy(table_hbm.at[IB[s]], B[s], G[s]).wait()

        def fire_write(j, s):
            off = pl.multiple_of(out_base + j * _C, 8)
            pltpu.async_copy(A[s], out_hbm.at[pl.ds(off, _C)], W[s])

        def wait_write(s):
            pltpu.make_async_copy(A[s], out_hbm.at[pl.ds(0, _C)], W[s]).wait()

        def add_rows(s):
            acc = A[s]
            rb = B[s]

            @plsc.parallel_loop(0, _C, unroll=4)
            def _(r):
                for q in range(_D // 16):
                    sl = pl.ds(q * 16, 16)
                    acc[r, sl] = acc[r, sl] + rb[r, sl]

        def phase(j, s):
            wait_gather(s)

            @pl.when(j <= n_chunks - 2)
            def _():
                wait_idx(s ^ 1)

                @pl.when(j >= 1)
                def _():
                    wait_write(s ^ 1)

                fire_gather(s ^ 1)

                @pl.when(j <= n_chunks - 3)
                def _():
                    fire_idx(j + 2, s)

            add_rows(s)
            fire_write(j, s)

        fire_idx(0, 0)
        wait_idx(0)
        fire_gather(0)
        fire_idx(1, 1)

        def pair(i, carry):
            phase(2 * i, 0)
            phase(2 * i + 1, 1)
            return carry

        lax.fori_loop(0, n_chunks // 2, pair, 0)
        if n_chunks % 2:
            phase(jnp.int32(n_chunks - 1), 0)
        wait_write(0)
        wait_write(1)

    return k(table, src, tgt_off)


def _mlp_body(dst_ref, g_ref, ea_ref, w1c_ref, b1_ref, w2_ref, b2_ref,
              out_ref):
    del dst_ref
    pre = jnp.dot(ea_ref[...], w1c_ref[...],
                  preferred_element_type=jnp.float32)
    h = jnp.maximum(pre + g_ref[...] + b1_ref[...], 0.0)
    out_ref[...] = jnp.dot(h, w2_ref[...],
                           preferred_element_type=jnp.float32) + b2_ref[...]


def _tc_mlp_slice(dest, gsum, edge_attr, w1c, b1, w2, b2, slice_k):
    blk0 = slice_k * (_EPK // _EB)
    in_specs = [
        pl.BlockSpec((_EB, _D), lambda i: (i, 0)),
        pl.BlockSpec((_EB, _EA_DIM), lambda i: (blk0 + i, 0)),
        pl.BlockSpec((_EA_DIM, _D), lambda i: (0, 0)),
        pl.BlockSpec((1, _D), lambda i: (0, 0)),
        pl.BlockSpec((_D, _D), lambda i: (0, 0)),
        pl.BlockSpec((1, _D), lambda i: (0, 0)),
    ]
    out_spec = pl.BlockSpec((_EB, _D), lambda i: (blk0 + i, 0))
    out_shape = jax.ShapeDtypeStruct((_N_EDGES, _D), jnp.float32)
    return pl.pallas_call(
        _mlp_body,
        grid=(_EPK // _EB,),
        in_specs=[pl.BlockSpec(memory_space=pl.ANY)] + in_specs,
        out_specs=out_spec,
        out_shape=out_shape,
        input_output_aliases={0: 0},
    )(dest, gsum, edge_attr, w1c, b1, w2, b2)


def kernel(x, edge_index, edge_attr, W1, b1, W2, b2):
    src = edge_index[0].astype(jnp.int32)
    tgt_off = edge_index[1].astype(jnp.int32) + _N_NODES
    w1ab = W1[: 2 * _D].reshape(2, _D, _D)
    w1c = W1[2 * _D:]
    b1r = b1.reshape(1, _D)
    b2r = b2.reshape(1, _D)
    table = _tc_tables(x, w1ab)
    dest = _alloc_dest()
    gsums = [_sc_gather_sum(table, src, tgt_off, k) for k in range(_K)]
    for k in range(_K):
        dest = _tc_mlp_slice(dest, gsums[k], edge_attr, w1c, b1r, W2, b2r, k)
    return dest


# speedup vs baseline: 1.1321x; 1.0142x over previous
import functools

import jax
import jax.numpy as jnp
from jax import lax
from jax.experimental import pallas as pl
from jax.experimental.pallas import tpu as pltpu
from jax.experimental.pallas import tpu_sc as plsc

_N_NODES = 10000
_N_EDGES = 320000
_D = 128
_EA_DIM = 16

_NC = 2
_NS = 16
_NW = _NC * _NS
_K = 5
_EPK = _N_EDGES // _K
_EDGES_PER_W = _EPK // _NW
_C = 200
_CHUNKS_PER_W = _EDGES_PER_W // _C
_EB = 8000


def _tc_tables(x, w1ab):
    def body(x_ref, w_ref, out_ref):
        out_ref[...] = jnp.dot(x_ref[...], w_ref[0],
                               preferred_element_type=jnp.float32)

    return pl.pallas_call(
        body,
        grid=(2,),
        in_specs=[
            pl.BlockSpec((_N_NODES, _D), lambda t: (0, 0)),
            pl.BlockSpec((1, _D, _D), lambda t: (t, 0, 0)),
        ],
        out_specs=pl.BlockSpec((_N_NODES, _D), lambda t: (t, 0)),
        out_shape=jax.ShapeDtypeStruct((2 * _N_NODES, _D), jnp.float32),
    )(x, w1ab)


def _alloc_dest():
    def body(dest_ref):
        del dest_ref

    return pl.pallas_call(
        body,
        out_specs=pl.BlockSpec(memory_space=pl.ANY),
        out_shape=jax.ShapeDtypeStruct((_N_EDGES, _D), jnp.float32),
    )()


def _sc_gather_sum(table, src, tgt_off, slice_k):
    mesh = plsc.VectorSubcoreMesh(core_axis_name="c", subcore_axis_name="s",
                                  num_cores=_NC, num_subcores=_NS)
    n_chunks = _CHUNKS_PER_W

    @functools.partial(
        pl.kernel,
        out_type=jax.ShapeDtypeStruct((_EPK, _D), jnp.float32),
        mesh=mesh,
        scratch_types=[
            pltpu.VMEM((_C,), jnp.int32),
            pltpu.VMEM((_C,), jnp.int32),
            pltpu.VMEM((_C,), jnp.int32),
            pltpu.VMEM((_C,), jnp.int32),
            pltpu.VMEM((_C, _D), jnp.float32),
            pltpu.VMEM((_C, _D), jnp.float32),
            pltpu.VMEM((_C, _D), jnp.float32),
            pltpu.VMEM((_C, _D), jnp.float32),
            pltpu.SemaphoreType.DMA,
            pltpu.SemaphoreType.DMA,
            pltpu.SemaphoreType.DMA,
            pltpu.SemaphoreType.DMA,
            pltpu.SemaphoreType.DMA,
            pltpu.SemaphoreType.DMA,
        ],
    )
    def k(table_hbm, src_hbm, tgt_hbm, out_hbm,
          ia0, ib0, ia1, ib1, a0, b0, a1, b1, g0, g1, s0, s1, w0, w1):
        wid = lax.axis_index("s") * _NC + lax.axis_index("c")
        in_base = slice_k * _EPK + wid * _EDGES_PER_W
        out_base = wid * _EDGES_PER_W
        IA = (ia0, ia1)
        IB = (ib0, ib1)
        A = (a0, a1)
        B = (b0, b1)
        G = (g0, g1)
        S = (s0, s1)
        W = (w0, w1)

        def fire_idx(j, s):
            off = pl.multiple_of(in_base + j * _C, 8)
            pltpu.async_copy(src_hbm.at[pl.ds(off, _C)], IA[s], S[s])
            pltpu.async_copy(tgt_hbm.at[pl.ds(off, _C)], IB[s], S[s])

        def wait_idx(s):
            pltpu.make_async_copy(src_hbm.at[pl.ds(0, _C)], IA[s], S[s]).wait()
            pltpu.make_async_copy(tgt_hbm.at[pl.ds(0, _C)], IB[s], S[s]).wait()

        def fire_gather(s):
            pltpu.async_copy(table_hbm.at[IA[s]], A[s], G[s])
            pltpu.async_copy(table_hbm.at[IB[s]], B[s], G[s])

        def wait_gather(s):
            pltpu.make_async_copy(table_hbm.at[IA[s]], A[s], G[s]).wait()
            pltpu.make_async_copy(table_hbm.at[IB[s]], B[s], G[s]).wait()

        def fire_write(j, s):
            off = pl.multiple_of(out_base + j * _C, 8)
            pltpu.async_copy(A[s], out_hbm.at[pl.ds(off, _C)], W[s])

        def wait_write(s):
            pltpu.make_async_copy(A[s], out_hbm.at[pl.ds(0, _C)], W[s]).wait()

        def add_rows(s):
            acc = A[s]
            rb = B[s]

            @plsc.parallel_loop(0, _C, unroll=4)
            def _(r):
                for q in range(_D // 16):
                    sl = pl.ds(q * 16, 16)
                    acc[r, sl] = acc[r, sl] + rb[r, sl]

        def phase(j, s):
            wait_gather(s)

            @pl.when(j <= n_chunks - 2)
            def _():
                wait_idx(s ^ 1)

                @pl.when(j >= 1)
                def _():
                    wait_write(s ^ 1)

                fire_gather(s ^ 1)

                @pl.when(j <= n_chunks - 3)
                def _():
                    fire_idx(j + 2, s)

            add_rows(s)
            fire_write(j, s)

        fire_idx(0, 0)
        wait_idx(0)
        fire_gather(0)
        fire_idx(1, 1)

        def pair(i, carry):
            phase(2 * i, 0)
            phase(2 * i + 1, 1)
            return carry

        lax.fori_loop(0, n_chunks // 2, pair, 0)
        if n_chunks % 2:
            phase(jnp.int32(n_chunks - 1), 0)
        wait_write(0)
        wait_write(1)

    return k(table, src, tgt_off)


def _mlp_body(dst_ref, g_ref, ea_ref, w1c_ref, b1_ref, w2_ref, b2_ref,
              out_ref):
    del dst_ref
    pre = jnp.dot(ea_ref[...], w1c_ref[...],
                  preferred_element_type=jnp.float32)
    h = jnp.maximum(pre + g_ref[...] + b1_ref[...], 0.0)
    out_ref[...] = jnp.dot(h, w2_ref[...],
                           preferred_element_type=jnp.float32) + b2_ref[...]


def _tc_mlp_slice(dest, gsum, edge_attr, w1c, b1, w2, b2, slice_k):
    blk0 = slice_k * (_EPK // _EB)
    in_specs = [
        pl.BlockSpec((_EB, _D), lambda i: (i, 0)),
        pl.BlockSpec((_EB, _EA_DIM), lambda i: (blk0 + i, 0)),
        pl.BlockSpec((_EA_DIM, _D), lambda i: (0, 0)),
        pl.BlockSpec((1, _D), lambda i: (0, 0)),
        pl.BlockSpec((_D, _D), lambda i: (0, 0)),
        pl.BlockSpec((1, _D), lambda i: (0, 0)),
    ]
    out_spec = pl.BlockSpec((_EB, _D), lambda i: (blk0 + i, 0))
    out_shape = jax.ShapeDtypeStruct((_N_EDGES, _D), jnp.float32)
    return pl.pallas_call(
        _mlp_body,
        grid=(_EPK // _EB,),
        in_specs=[pl.BlockSpec(memory_space=pl.ANY)] + in_specs,
        out_specs=out_spec,
        out_shape=out_shape,
        input_output_aliases={0: 0},
    )(dest, gsum, edge_attr, w1c, b1, w2, b2)


def kernel(x, edge_index, edge_attr, W1, b1, W2, b2):
    src = edge_index[0].astype(jnp.int32)
    tgt_off = edge_index[1].astype(jnp.int32) + _N_NODES
    w1ab = W1[: 2 * _D].reshape(2, _D, _D)
    w1c = W1[2 * _D:]
    b1r = b1.reshape(1, _D)
    b2r = b2.reshape(1, _D)
    table = _tc_tables(x, w1ab)
    dest = _alloc_dest()
    gsums = [_sc_gather_sum(table, src, tgt_off, k) for k in range(_K)]
    for k in range(_K):
        dest = _tc_mlp_slice(dest, gsums[k], edge_attr, w1c, b1r, W2, b2r, k)
    return dest
